# baseline (speedup 1.0000x reference)
import jax
import jax.numpy as jnp
from jax import lax
from jax.experimental import pallas as pl
from jax.experimental.pallas import tpu as pltpu
from jax.experimental.pallas import tpu_sc as plsc

_N = 10000
_D = 128
_DH = 4
_NT = 32
_E = 320000
_L = 10
_NC = 2
_NS = 16
_EPAD = 327680
_R128 = _EPAD // 128
_SB = 8
_NSB = _R128 // _SB
_NPAD = 10240
_W = _NPAD * _DH
_FCH = 4096
_NFC = _W // _FCH


def _body(row_h, col_h, w_h, xr_h, mw_h, out_h,
          hu, hv, rowi, coli, wts, fo, mwv, semi):
    c = lax.axis_index("c")
    s = lax.axis_index("s")
    w = s * _NC + c
    obase = w * _W

    pltpu.sync_copy(mw_h, mwv)
    wv_all = mwv[pl.ds(0, 16)]

    def _lane(vec, idx):
        return vec.at[jnp.full((16,), idx, jnp.int32)].get(
            mode="promise_in_bounds")

    w0 = wv_all[0]

    pltpu.sync_copy(xr_h.at[pl.ds(obase, _W)], hu)

    def _initk(k, carry):
        off = k * _FCH

        def _sc(r, cc):
            sl = pl.ds(off + r * 16, 16)
            fo[pl.ds(r * 16, 16)] = hu[sl] * w0
            return cc
        lax.fori_loop(0, _FCH // 16, _sc, 0)
        pltpu.sync_copy(fo, out_h.at[pl.ds(obase + off, _FCH)])
        return carry
    lax.fori_loop(0, _NFC, _initk, 0)

    def _zero(buf):
        def _zk(r, cc):
            base = r * 128
            for k in range(8):
                buf[pl.ds(base + k * 16, 16)] = jnp.zeros((16,), jnp.float32)
            return cc
        lax.fori_loop(0, _W // 128, _zk, 0)
    _zero(hv)

    def _edge_row(hsrc, hdst, jrow):
        for g in range(8):
            sl = pl.ds(g * 16, 16)
            col16 = coli[jrow, sl] * 4
            row16 = rowi[jrow, sl] * 4
            w16 = wts[jrow, sl]
            for f in range(_DH):
                gv = plsc.load_gather(hsrc, [col16 + f])
                plsc.addupdate_scatter(hdst, [row16 + f], gv * w16)

    def _hop(hsrc, hdst, l):
        pltpu.sync_copy(row_h.at[pl.ds(0, _SB)], rowi.at[pl.ds(0, _SB)])
        pltpu.sync_copy(col_h.at[pl.ds(0, _SB)], coli.at[pl.ds(0, _SB)])
        pltpu.sync_copy(w_h.at[pl.ds(0, _SB)], wts.at[pl.ds(0, _SB)])

        def _sbody(it, c2):
            par = lax.rem(it, 2)
            base_cur = par * _SB
            base_nxt = (1 - par) * _SB
            src = lax.rem(it + 1, _NSB) * _SB
            pltpu.async_copy(
                row_h.at[pl.ds(src, _SB)], rowi.at[pl.ds(base_nxt, _SB)],
                semi)
            pltpu.async_copy(
                col_h.at[pl.ds(src, _SB)], coli.at[pl.ds(base_nxt, _SB)],
                semi)
            pltpu.async_copy(
                w_h.at[pl.ds(src, _SB)], wts.at[pl.ds(base_nxt, _SB)], semi)
            for pos in range(_SB):
                _edge_row(hsrc, hdst, base_cur + pos)
            pltpu.make_async_copy(
                row_h.at[pl.ds(src, _SB)], rowi.at[pl.ds(base_nxt, _SB)],
                semi).wait()
            pltpu.make_async_copy(
                col_h.at[pl.ds(src, _SB)], coli.at[pl.ds(base_nxt, _SB)],
                semi).wait()
            pltpu.make_async_copy(
                w_h.at[pl.ds(src, _SB)], wts.at[pl.ds(base_nxt, _SB)],
                semi).wait()
            return c2
        lax.fori_loop(0, _NSB, _sbody, 0)

        wl = _lane(wv_all, l)

        def _fl(k, c2):
            off = k * _FCH
            pltpu.sync_copy(out_h.at[pl.ds(obase + off, _FCH)], fo)

            def _ac(r, c3):
                sl = pl.ds(off + r * 16, 16)
                so = pl.ds(r * 16, 16)
                fo[so] = fo[so] + hdst[sl] * wl
                return c3
            lax.fori_loop(0, _FCH // 16, _ac, 0)
            pltpu.sync_copy(fo, out_h.at[pl.ds(obase + off, _FCH)])
            return c2
        lax.fori_loop(0, _NFC, _fl, 0)
        _zero(hsrc)

    def _pair(i, carry):
        _hop(hu, hv, 2 * i + 1)
        _hop(hv, hu, 2 * i + 2)
        return carry
    lax.fori_loop(0, _L // 2, _pair, 0)


def _make_call():
    mesh = plsc.VectorSubcoreMesh(
        core_axis_name="c", subcore_axis_name="s",
        num_cores=_NC, num_subcores=_NS)
    return pl.kernel(
        _body,
        out_type=[
            jax.ShapeDtypeStruct((_NT * _W,), jnp.float32),
        ],
        mesh=mesh,
        compiler_params=pltpu.CompilerParams(use_tc_tiling_on_sc=False,
                                             needs_layout_passes=False),
        scratch_types=[
            pltpu.VMEM((_W,), jnp.float32),
            pltpu.VMEM((_W,), jnp.float32),
            pltpu.VMEM((2 * _SB, 128), jnp.int32),
            pltpu.VMEM((2 * _SB, 128), jnp.int32),
            pltpu.VMEM((2 * _SB, 128), jnp.float32),
            pltpu.VMEM((_FCH,), jnp.float32),
            pltpu.VMEM((16,), jnp.float32),
            pltpu.SemaphoreType.DMA,
        ],
    )


def kernel(edge_index, edge_weight, X_manifold, manifold_weights):
    row = edge_index[0]
    col = edge_index[1]
    pad = _EPAD - _E
    zi = jnp.zeros((pad,), jnp.int32)
    row2 = jnp.concatenate([row, zi]).reshape(_R128, 128)
    col2 = jnp.concatenate([col, zi]).reshape(_R128, 128)
    w2 = jnp.concatenate([edge_weight, jnp.zeros((pad,), jnp.float32)]
                         ).reshape(_R128, 128)
    xp = jnp.pad(X_manifold, ((0, _NPAD - _N), (0, 0)))
    xr = xp.reshape(_NPAD, _NT, _DH).transpose(1, 0, 2).reshape(-1)
    mw = jnp.concatenate([manifold_weights,
                          jnp.zeros((16 - _L - 1,), jnp.float32)])
    (out,) = _make_call()(row2, col2, w2, xr, mw)
    out = out.reshape(_NT, _NPAD, _DH).transpose(1, 0, 2).reshape(_NPAD, _D)
    return out[:_N]


# speedup vs baseline: 2.1767x; 2.1767x over previous
import jax
import jax.numpy as jnp
from jax import lax
from jax.experimental import pallas as pl
from jax.experimental.pallas import tpu as pltpu
from jax.experimental.pallas import tpu_sc as plsc

_N = 10000
_D = 128
_DH = 4
_NT = 32
_E = 320000
_L = 10
_NC = 2
_NS = 16
_EPAD = 327680
_R128 = _EPAD // 128
_SB = 8
_NSB = _R128 // _SB
_NPAD = 10240
_W = _NPAD * _DH
_FCH = 4096
_NFC = _W // _FCH


def _body(row_h, col_h, w_h, xr_h, mw_h, out_h,
          hu, hv, rowi, coli, wts, fo, mwv, semi):
    c = lax.axis_index("c")
    s = lax.axis_index("s")
    w = s * _NC + c
    obase = w * _W

    pltpu.sync_copy(mw_h, mwv)
    wv_all = mwv[pl.ds(0, 16)]

    def _lane(vec, idx):
        return vec.at[jnp.full((16,), idx, jnp.int32)].get(
            mode="promise_in_bounds")

    w0 = wv_all[0]

    pltpu.sync_copy(xr_h.at[pl.ds(obase, _W)], hu)

    def _initk(k, carry):
        off = k * _FCH

        def _sc(r, cc):
            sl = pl.ds(off + r * 16, 16)
            fo[pl.ds(r * 16, 16)] = hu[sl] * w0
            return cc
        lax.fori_loop(0, _FCH // 16, _sc, 0)
        pltpu.sync_copy(fo, out_h.at[pl.ds(obase + off, _FCH)])
        return carry
    lax.fori_loop(0, _NFC, _initk, 0)

    def _zero(buf):
        def _zk(r, cc):
            base = r * 128
            for k in range(8):
                buf[pl.ds(base + k * 16, 16)] = jnp.zeros((16,), jnp.float32)
            return cc
        lax.fori_loop(0, _W // 128, _zk, 0)
    _zero(hv)

    def _edge_row(hsrc, hdst, jrow):
        for g in range(8):
            sl = pl.ds(g * 16, 16)
            col16 = coli[jrow, sl]
            row16 = rowi[jrow, sl]
            w16 = wts[jrow, sl]
            vals = [plsc.load_gather(hsrc, [col16 + f * _NPAD]) * w16
                    for f in range(_DH)]
            for f in range(_DH):
                plsc.addupdate_scatter(hdst, [row16 + f * _NPAD], vals[f])

    def _hop(hsrc, hdst, l):
        pltpu.sync_copy(row_h.at[pl.ds(0, _SB)], rowi.at[pl.ds(0, _SB)])
        pltpu.sync_copy(col_h.at[pl.ds(0, _SB)], coli.at[pl.ds(0, _SB)])
        pltpu.sync_copy(w_h.at[pl.ds(0, _SB)], wts.at[pl.ds(0, _SB)])

        def _sbody(it, c2):
            par = lax.rem(it, 2)
            base_cur = par * _SB
            base_nxt = (1 - par) * _SB
            src = lax.rem(it + 1, _NSB) * _SB
            pltpu.async_copy(
                row_h.at[pl.ds(src, _SB)], rowi.at[pl.ds(base_nxt, _SB)],
                semi)
            pltpu.async_copy(
                col_h.at[pl.ds(src, _SB)], coli.at[pl.ds(base_nxt, _SB)],
                semi)
            pltpu.async_copy(
                w_h.at[pl.ds(src, _SB)], wts.at[pl.ds(base_nxt, _SB)], semi)
            for pos in range(_SB):
                _edge_row(hsrc, hdst, base_cur + pos)
            pltpu.make_async_copy(
                row_h.at[pl.ds(src, _SB)], rowi.at[pl.ds(base_nxt, _SB)],
                semi).wait()
            pltpu.make_async_copy(
                col_h.at[pl.ds(src, _SB)], coli.at[pl.ds(base_nxt, _SB)],
                semi).wait()
            pltpu.make_async_copy(
                w_h.at[pl.ds(src, _SB)], wts.at[pl.ds(base_nxt, _SB)],
                semi).wait()
            return c2
        lax.fori_loop(0, _NSB, _sbody, 0)

        wl = _lane(wv_all, l)

        def _fl(k, c2):
            off = k * _FCH
            pltpu.sync_copy(out_h.at[pl.ds(obase + off, _FCH)], fo)

            def _ac(r, c3):
                sl = pl.ds(off + r * 16, 16)
                so = pl.ds(r * 16, 16)
                fo[so] = fo[so] + hdst[sl] * wl
                return c3
            lax.fori_loop(0, _FCH // 16, _ac, 0)
            pltpu.sync_copy(fo, out_h.at[pl.ds(obase + off, _FCH)])
            return c2
        lax.fori_loop(0, _NFC, _fl, 0)
        _zero(hsrc)

    def _pair(i, carry):
        _hop(hu, hv, 2 * i + 1)
        _hop(hv, hu, 2 * i + 2)
        return carry
    lax.fori_loop(0, _L // 2, _pair, 0)


def _make_call():
    mesh = plsc.VectorSubcoreMesh(
        core_axis_name="c", subcore_axis_name="s",
        num_cores=_NC, num_subcores=_NS)
    return pl.kernel(
        _body,
        out_type=[
            jax.ShapeDtypeStruct((_NT * _W,), jnp.float32),
        ],
        mesh=mesh,
        compiler_params=pltpu.CompilerParams(use_tc_tiling_on_sc=False,
                                             needs_layout_passes=False),
        scratch_types=[
            pltpu.VMEM((_W,), jnp.float32),
            pltpu.VMEM((_W,), jnp.float32),
            pltpu.VMEM((2 * _SB, 128), jnp.int32),
            pltpu.VMEM((2 * _SB, 128), jnp.int32),
            pltpu.VMEM((2 * _SB, 128), jnp.float32),
            pltpu.VMEM((_FCH,), jnp.float32),
            pltpu.VMEM((16,), jnp.float32),
            pltpu.SemaphoreType.DMA,
        ],
    )


def kernel(edge_index, edge_weight, X_manifold, manifold_weights):
    row = edge_index[0]
    col = edge_index[1]
    pad = _EPAD - _E
    zi = jnp.zeros((pad,), jnp.int32)
    row2 = jnp.concatenate([row, zi]).reshape(_R128, 128)
    col2 = jnp.concatenate([col, zi]).reshape(_R128, 128)
    w2 = jnp.concatenate([edge_weight, jnp.zeros((pad,), jnp.float32)]
                         ).reshape(_R128, 128)
    xp = jnp.pad(X_manifold, ((0, _NPAD - _N), (0, 0)))
    xr = xp.reshape(_NPAD, _NT, _DH).transpose(1, 2, 0).reshape(-1)
    mw = jnp.concatenate([manifold_weights,
                          jnp.zeros((16 - _L - 1,), jnp.float32)])
    (out,) = _make_call()(row2, col2, w2, xr, mw)
    out = out.reshape(_NT, _DH, _NPAD).transpose(2, 0, 1).reshape(_NPAD, _D)
    return out[:_N]


# speedup vs baseline: 3.9505x; 1.8149x over previous
import jax
import jax.numpy as jnp
from jax import lax
from jax.experimental import pallas as pl
from jax.experimental.pallas import tpu as pltpu
from jax.experimental.pallas import tpu_sc as plsc

_N = 10000
_D = 128
_DH = 64
_E = 320000
_L = 10
_NC = 2
_NS = 16
_EPAD = 327680
_R128 = _EPAD // 128
_RPT = _R128 // _NS
_SB = 8
_NSB = _RPT // _SB
_NPAD = 10240
_STRIPE = _NPAD // _NS
_FB = 64
_NFC = _STRIPE // _FB
_NV = _DH // 16


def _body(row_h, col_h, w_h, xc_h, mw_h, out_h, hbuf_h,
          acc_sh, hcur_sh, rowi, coli, wts, cadj, gb0, gb1, gb2, gb3,
          fa, fw, mwv,
          semg0, semg1, semg2, semg3, sems0, sems1, sems2, sems3, semi):
    gbs = (gb0, gb1, gb2, gb3)
    semg = (semg0, semg1, semg2, semg3)
    sems = (sems0, sems1, sems2, sems3)
    c = lax.axis_index("c")
    s = lax.axis_index("s")
    cN = c * _NPAD
    ebase = s * _RPT

    pltpu.sync_copy(mw_h, mwv)
    wv_all = mwv[pl.ds(0, 16)]

    def _lane(vec, idx):
        return vec.at[jnp.full((16,), idx, jnp.int32)].get(
            mode="promise_in_bounds")

    w0 = wv_all[0]

    def _init(k, carry):
        off = s * _STRIPE + k * _FB
        pltpu.sync_copy(xc_h.at[pl.ds(cN + off, _FB)], fa)
        pltpu.sync_copy(fa, hcur_sh.at[pl.ds(off, _FB)])
        pltpu.sync_copy(fa, hbuf_h.at[pl.ds(cN + off, _FB)])

        def _sc(r, cc):
            for f in range(_NV):
                sl = pl.ds(f * 16, 16)
                fw[r, sl] = fa[r, sl] * w0
            return cc
        lax.fori_loop(0, _FB, _sc, 0)
        pltpu.sync_copy(fw, out_h.at[pl.ds(cN + off, _FB)])

        def _zf(r, cc):
            for f in range(_NV):
                fw[r, pl.ds(f * 16, 16)] = jnp.zeros((16,), jnp.float32)
            return cc
        lax.fori_loop(0, _FB, _zf, 0)
        pltpu.sync_copy(fw, acc_sh.at[pl.ds(off, _FB)])
        return carry
    lax.fori_loop(0, _NFC, _init, 0)
    plsc.subcore_barrier()

    def _mult(gb, jrow):
        for g in range(128 // 16):
            wv = wts[jrow, pl.ds(g * 16, 16)]
            for t in range(16):
                e = g * 16 + t
                w = wv[t]
                for f in range(_NV):
                    sl = pl.ds(f * 16, 16)
                    gb[e, sl] = gb[e, sl] * w

    def _issue_gather(p, jrow, hbm):
        for f in range(128 // 16):
            sl = pl.ds(f * 16, 16)
            if hbm:
                cadj[p, sl] = coli[jrow, sl] + cN
            else:
                cadj[p, sl] = coli[jrow, sl]
        if hbm:
            pltpu.async_copy(hbuf_h.at[cadj.at[p]], gbs[p], semg[p])
        else:
            pltpu.async_copy(hcur_sh.at[cadj.at[p]], gbs[p], semg[p])

    def _hop(l, carry):
        pltpu.sync_copy(row_h.at[pl.ds(ebase, _SB)], rowi.at[pl.ds(0, _SB)])
        pltpu.sync_copy(col_h.at[pl.ds(ebase, _SB)], coli.at[pl.ds(0, _SB)])
        pltpu.sync_copy(w_h.at[pl.ds(ebase, _SB)], wts.at[pl.ds(0, _SB)])
        _issue_gather(0, 0, False)
        _issue_gather(1, 1, True)

        def _sbody(it, c2):
            par = lax.rem(it, 2)
            base_cur = par * _SB
            base_nxt = (1 - par) * _SB
            nsb = lax.rem(it + 1, _NSB)
            src = ebase + nsb * _SB

            @pl.when(it > 0)
            def _drain_prev():
                pltpu.make_async_copy(
                    gbs[2], acc_sh.at[rowi.at[0]], sems[2]).wait()
                pltpu.make_async_copy(
                    gbs[3], acc_sh.at[rowi.at[0]], sems[3]).wait()
            pltpu.async_copy(
                row_h.at[pl.ds(src, _SB)], rowi.at[pl.ds(base_nxt, _SB)],
                semi)
            pltpu.async_copy(
                col_h.at[pl.ds(src, _SB)], coli.at[pl.ds(base_nxt, _SB)],
                semi)
            pltpu.async_copy(
                w_h.at[pl.ds(src, _SB)], wts.at[pl.ds(base_nxt, _SB)], semi)
            for pos in range(_SB):
                p = pos % 4
                p2 = (pos + 2) % 4
                jrow = base_cur + pos
                if p % 2 == 0:
                    pltpu.make_async_copy(
                        hcur_sh.at[cadj.at[p]], gbs[p], semg[p]).wait()
                else:
                    pltpu.make_async_copy(
                        hbuf_h.at[cadj.at[p]], gbs[p], semg[p]).wait()
                if pos >= 2:
                    pltpu.make_async_copy(
                        gbs[p2], acc_sh.at[rowi.at[0]], sems[p2]).wait()
                if pos < _SB - 2:
                    _issue_gather(p2, base_cur + pos + 2, pos % 2 == 1)
                else:
                    if pos == _SB - 2:
                        pltpu.make_async_copy(
                            row_h.at[pl.ds(src, _SB)],
                            rowi.at[pl.ds(base_nxt, _SB)], semi).wait()
                        pltpu.make_async_copy(
                            col_h.at[pl.ds(src, _SB)],
                            coli.at[pl.ds(base_nxt, _SB)], semi).wait()
                        pltpu.make_async_copy(
                            w_h.at[pl.ds(src, _SB)],
                            wts.at[pl.ds(base_nxt, _SB)], semi).wait()
                    _issue_gather(p2, base_nxt + (pos - (_SB - 2)),
                                  pos % 2 == 1)
                _mult(gbs[p], jrow)
                pltpu.async_copy(
                    gbs[p], acc_sh.at[rowi.at[jrow]], sems[p], add=True)
            return c2
        lax.fori_loop(0, _NSB, _sbody, 0)
        pltpu.make_async_copy(hcur_sh.at[cadj.at[0]], gbs[0], semg[0]).wait()
        pltpu.make_async_copy(hbuf_h.at[cadj.at[1]], gbs[1], semg[1]).wait()
        pltpu.make_async_copy(gbs[2], acc_sh.at[rowi.at[0]], sems[2]).wait()
        pltpu.make_async_copy(gbs[3], acc_sh.at[rowi.at[0]], sems[3]).wait()
        plsc.subcore_barrier()

        wl = _lane(wv_all, l)

        def _fl(k, c2):
            off = s * _STRIPE + k * _FB
            pltpu.sync_copy(acc_sh.at[pl.ds(off, _FB)], fa)
            pltpu.sync_copy(fa, hcur_sh.at[pl.ds(off, _FB)])
            pltpu.sync_copy(fa, hbuf_h.at[pl.ds(cN + off, _FB)])
            pltpu.sync_copy(out_h.at[pl.ds(cN + off, _FB)], fw)

            def _ac(r, c3):
                for f in range(_NV):
                    sl = pl.ds(f * 16, 16)
                    fw[r, sl] = fw[r, sl] + fa[r, sl] * wl
                return c3
            lax.fori_loop(0, _FB, _ac, 0)
            pltpu.sync_copy(fw, out_h.at[pl.ds(cN + off, _FB)])

            def _zf(r, c3):
                for f in range(_NV):
                    fw[r, pl.ds(f * 16, 16)] = jnp.zeros((16,), jnp.float32)
                return c3
            lax.fori_loop(0, _FB, _zf, 0)
            pltpu.sync_copy(fw, acc_sh.at[pl.ds(off, _FB)])
            return c2
        lax.fori_loop(0, _NFC, _fl, 0)
        plsc.subcore_barrier()
        return carry
    lax.fori_loop(1, _L + 1, _hop, 0)


def _make_call():
    mesh = plsc.VectorSubcoreMesh(
        core_axis_name="c", subcore_axis_name="s",
        num_cores=_NC, num_subcores=_NS)
    return pl.kernel(
        _body,
        out_type=[
            jax.ShapeDtypeStruct((2 * _NPAD, _DH), jnp.float32),
            jax.ShapeDtypeStruct((2 * _NPAD, _DH), jnp.float32),
        ],
        mesh=mesh,
        compiler_params=pltpu.CompilerParams(use_tc_tiling_on_sc=False),
        scratch_types=[
            pltpu.VMEM_SHARED((_NPAD, _DH), jnp.float32),
            pltpu.VMEM_SHARED((_NPAD, _DH), jnp.float32),
            pltpu.VMEM((2 * _SB, 128), jnp.int32),
            pltpu.VMEM((2 * _SB, 128), jnp.int32),
            pltpu.VMEM((2 * _SB, 128), jnp.float32),
            pltpu.VMEM((4, 128), jnp.int32),
            pltpu.VMEM((128, _DH), jnp.float32),
            pltpu.VMEM((128, _DH), jnp.float32),
            pltpu.VMEM((128, _DH), jnp.float32),
            pltpu.VMEM((128, _DH), jnp.float32),
            pltpu.VMEM((_FB, _DH), jnp.float32),
            pltpu.VMEM((_FB, _DH), jnp.float32),
            pltpu.VMEM((16,), jnp.float32),
            pltpu.SemaphoreType.DMA,
            pltpu.SemaphoreType.DMA,
            pltpu.SemaphoreType.DMA,
            pltpu.SemaphoreType.DMA,
            pltpu.SemaphoreType.DMA,
            pltpu.SemaphoreType.DMA,
            pltpu.SemaphoreType.DMA,
            pltpu.SemaphoreType.DMA,
            pltpu.SemaphoreType.DMA,
        ],
    )


def kernel(edge_index, edge_weight, X_manifold, manifold_weights):
    row = edge_index[0]
    col = edge_index[1]
    pad = _EPAD - _E
    zi = jnp.zeros((pad,), jnp.int32)
    row2 = jnp.concatenate([row, zi]).reshape(_R128, 128)
    col2 = jnp.concatenate([col, zi]).reshape(_R128, 128)
    w2 = jnp.concatenate([edge_weight, jnp.zeros((pad,), jnp.float32)]
                         ).reshape(_R128, 128)
    rpad = ((0, _NPAD - _N), (0, 0))
    xc = jnp.concatenate([jnp.pad(X_manifold[:, :_DH], rpad),
                          jnp.pad(X_manifold[:, _DH:], rpad)], axis=0)
    mw = jnp.concatenate([manifold_weights,
                          jnp.zeros((16 - _L - 1,), jnp.float32)])
    out, _ = _make_call()(row2, col2, w2, xc, mw)
    return jnp.concatenate([out[:_N], out[_NPAD:_NPAD + _N]], axis=1)


# speedup vs baseline: 4.0853x; 1.0341x over previous
import jax
import jax.numpy as jnp
from jax import lax
from jax.experimental import pallas as pl
from jax.experimental.pallas import tpu as pltpu
from jax.experimental.pallas import tpu_sc as plsc

_N = 10000
_D = 128
_DH = 64
_E = 320000
_L = 10
_NC = 2
_NS = 16
_EPAD = 327680
_R128 = _EPAD // 128
_RPT = _R128 // _NS
_SB = 8
_NSB = _RPT // _SB
_NPAD = 10240
_STRIPE = _NPAD // _NS
_FB = 64
_NFC = _STRIPE // _FB
_NV = _DH // 16


def _body(row_h, col_h, w_h, xc_h, mw_h, out_h,
          acc_sh, hcur_sh, rowi, coli, wts, cadj, gb0, gb1, gb2, gb3,
          fa, fw, mwv,
          semg0, semg1, semg2, semg3, sems0, sems1, sems2, sems3, semi):
    gbs = (gb0, gb1, gb2, gb3)
    semg = (semg0, semg1, semg2, semg3)
    sems = (sems0, sems1, sems2, sems3)
    c = lax.axis_index("c")
    s = lax.axis_index("s")
    cN = c * _NPAD
    ebase = s * _RPT

    pltpu.sync_copy(mw_h, mwv)
    wv_all = mwv[pl.ds(0, 16)]

    def _lane(vec, idx):
        return vec.at[jnp.full((16,), idx, jnp.int32)].get(
            mode="promise_in_bounds")

    w0 = wv_all[0]

    def _init(k, carry):
        off = s * _STRIPE + k * _FB
        pltpu.sync_copy(xc_h.at[pl.ds(cN + off, _FB)], fa)
        pltpu.sync_copy(fa, hcur_sh.at[pl.ds(off, _FB)])

        def _sc(r, cc):
            for f in range(_NV):
                sl = pl.ds(f * 16, 16)
                fw[r, sl] = fa[r, sl] * w0
            return cc
        lax.fori_loop(0, _FB, _sc, 0)
        pltpu.sync_copy(fw, out_h.at[pl.ds(cN + off, _FB)])

        def _zf(r, cc):
            for f in range(_NV):
                fw[r, pl.ds(f * 16, 16)] = jnp.zeros((16,), jnp.float32)
            return cc
        lax.fori_loop(0, _FB, _zf, 0)
        pltpu.sync_copy(fw, acc_sh.at[pl.ds(off, _FB)])
        return carry
    lax.fori_loop(0, _NFC, _init, 0)
    plsc.subcore_barrier()

    def _mult(gb, jrow):
        for g in range(128 // 16):
            wv = wts[jrow, pl.ds(g * 16, 16)]
            for t in range(16):
                e = g * 16 + t
                w = wv[t]
                for f in range(_NV):
                    sl = pl.ds(f * 16, 16)
                    gb[e, sl] = gb[e, sl] * w

    def _issue_gather(p, jrow):
        for f in range(128 // 16):
            sl = pl.ds(f * 16, 16)
            cadj[p, sl] = coli[jrow, sl]
        pltpu.async_copy(hcur_sh.at[cadj.at[p]], gbs[p], semg[p])

    def _hop(l, carry):
        pltpu.sync_copy(row_h.at[pl.ds(ebase, _SB)], rowi.at[pl.ds(0, _SB)])
        pltpu.sync_copy(col_h.at[pl.ds(ebase, _SB)], coli.at[pl.ds(0, _SB)])
        pltpu.sync_copy(w_h.at[pl.ds(ebase, _SB)], wts.at[pl.ds(0, _SB)])
        _issue_gather(0, 0)
        _issue_gather(1, 1)

        def _sbody(it, c2):
            par = lax.rem(it, 2)
            base_cur = par * _SB
            base_nxt = (1 - par) * _SB
            nsb = lax.rem(it + 1, _NSB)
            src = ebase + nsb * _SB

            @pl.when(it > 0)
            def _drain_prev():
                pltpu.make_async_copy(
                    gbs[2], acc_sh.at[rowi.at[0]], sems[2]).wait()
                pltpu.make_async_copy(
                    gbs[3], acc_sh.at[rowi.at[0]], sems[3]).wait()
            pltpu.async_copy(
                row_h.at[pl.ds(src, _SB)], rowi.at[pl.ds(base_nxt, _SB)],
                semi)
            pltpu.async_copy(
                col_h.at[pl.ds(src, _SB)], coli.at[pl.ds(base_nxt, _SB)],
                semi)
            pltpu.async_copy(
                w_h.at[pl.ds(src, _SB)], wts.at[pl.ds(base_nxt, _SB)], semi)
            for pos in range(_SB):
                p = pos % 4
                p2 = (pos + 2) % 4
                jrow = base_cur + pos
                pltpu.make_async_copy(
                    hcur_sh.at[cadj.at[p]], gbs[p], semg[p]).wait()
                if pos >= 2:
                    pltpu.make_async_copy(
                        gbs[p2], acc_sh.at[rowi.at[0]], sems[p2]).wait()
                if pos < _SB - 2:
                    _issue_gather(p2, base_cur + pos + 2)
                else:
                    if pos == _SB - 2:
                        pltpu.make_async_copy(
                            row_h.at[pl.ds(src, _SB)],
                            rowi.at[pl.ds(base_nxt, _SB)], semi).wait()
                        pltpu.make_async_copy(
                            col_h.at[pl.ds(src, _SB)],
                            coli.at[pl.ds(base_nxt, _SB)], semi).wait()
                        pltpu.make_async_copy(
                            w_h.at[pl.ds(src, _SB)],
                            wts.at[pl.ds(base_nxt, _SB)], semi).wait()
                    _issue_gather(p2, base_nxt + (pos - (_SB - 2)))
                _mult(gbs[p], jrow)
                pltpu.async_copy(
                    gbs[p], acc_sh.at[rowi.at[jrow]], sems[p], add=True)
            return c2
        lax.fori_loop(0, _NSB, _sbody, 0)
        pltpu.make_async_copy(hcur_sh.at[cadj.at[0]], gbs[0], semg[0]).wait()
        pltpu.make_async_copy(hcur_sh.at[cadj.at[1]], gbs[1], semg[1]).wait()
        pltpu.make_async_copy(gbs[2], acc_sh.at[rowi.at[0]], sems[2]).wait()
        pltpu.make_async_copy(gbs[3], acc_sh.at[rowi.at[0]], sems[3]).wait()
        plsc.subcore_barrier()

        wl = _lane(wv_all, l)

        def _fl(k, c2):
            off = s * _STRIPE + k * _FB
            pltpu.sync_copy(acc_sh.at[pl.ds(off, _FB)], fa)
            pltpu.sync_copy(fa, hcur_sh.at[pl.ds(off, _FB)])
            pltpu.sync_copy(out_h.at[pl.ds(cN + off, _FB)], fw)

            def _ac(r, c3):
                for f in range(_NV):
                    sl = pl.ds(f * 16, 16)
                    fw[r, sl] = fw[r, sl] + fa[r, sl] * wl
                return c3
            lax.fori_loop(0, _FB, _ac, 0)
            pltpu.sync_copy(fw, out_h.at[pl.ds(cN + off, _FB)])

            def _zf(r, c3):
                for f in range(_NV):
                    fw[r, pl.ds(f * 16, 16)] = jnp.zeros((16,), jnp.float32)
                return c3
            lax.fori_loop(0, _FB, _zf, 0)
            pltpu.sync_copy(fw, acc_sh.at[pl.ds(off, _FB)])
            return c2
        lax.fori_loop(0, _NFC, _fl, 0)
        plsc.subcore_barrier()
        return carry
    lax.fori_loop(1, _L + 1, _hop, 0)


def _make_call():
    mesh = plsc.VectorSubcoreMesh(
        core_axis_name="c", subcore_axis_name="s",
        num_cores=_NC, num_subcores=_NS)
    return pl.kernel(
        _body,
        out_type=[
            jax.ShapeDtypeStruct((2 * _NPAD, _DH), jnp.float32),
        ],
        mesh=mesh,
        compiler_params=pltpu.CompilerParams(use_tc_tiling_on_sc=False),
        scratch_types=[
            pltpu.VMEM_SHARED((_NPAD, _DH), jnp.float32),
            pltpu.VMEM_SHARED((_NPAD, _DH), jnp.float32),
            pltpu.VMEM((2 * _SB, 128), jnp.int32),
            pltpu.VMEM((2 * _SB, 128), jnp.int32),
            pltpu.VMEM((2 * _SB, 128), jnp.float32),
            pltpu.VMEM((4, 128), jnp.int32),
            pltpu.VMEM((128, _DH), jnp.float32),
            pltpu.VMEM((128, _DH), jnp.float32),
            pltpu.VMEM((128, _DH), jnp.float32),
            pltpu.VMEM((128, _DH), jnp.float32),
            pltpu.VMEM((_FB, _DH), jnp.float32),
            pltpu.VMEM((_FB, _DH), jnp.float32),
            pltpu.VMEM((16,), jnp.float32),
            pltpu.SemaphoreType.DMA,
            pltpu.SemaphoreType.DMA,
            pltpu.SemaphoreType.DMA,
            pltpu.SemaphoreType.DMA,
            pltpu.SemaphoreType.DMA,
            pltpu.SemaphoreType.DMA,
            pltpu.SemaphoreType.DMA,
            pltpu.SemaphoreType.DMA,
            pltpu.SemaphoreType.DMA,
        ],
    )


def kernel(edge_index, edge_weight, X_manifold, manifold_weights):
    row = edge_index[0]
    col = edge_index[1]
    pad = _EPAD - _E
    zi = jnp.zeros((pad,), jnp.int32)
    row2 = jnp.concatenate([row, zi]).reshape(_R128, 128)
    col2 = jnp.concatenate([col, zi]).reshape(_R128, 128)
    w2 = jnp.concatenate([edge_weight, jnp.zeros((pad,), jnp.float32)]
                         ).reshape(_R128, 128)
    rpad = ((0, _NPAD - _N), (0, 0))
    xc = jnp.concatenate([jnp.pad(X_manifold[:, :_DH], rpad),
                          jnp.pad(X_manifold[:, _DH:], rpad)], axis=0)
    mw = jnp.concatenate([manifold_weights,
                          jnp.zeros((16 - _L - 1,), jnp.float32)])
    (out,) = _make_call()(row2, col2, w2, xc, mw)
    return jnp.concatenate([out[:_N], out[_NPAD:_NPAD + _N]], axis=1)
